# baseline (speedup 1.0000x reference)
import functools

import jax
import jax.numpy as jnp
from jax import lax
from jax.experimental import pallas as pl
from jax.experimental.pallas import tpu as pltpu
from jax.experimental.pallas import tpu_sc as plsc

B = 16384
V = 1000
D = 1000
NC = 2
NS = 16
NW = NC * NS
BPW = B // NW
CHUNK = 32
NCH = BPW // CHUNK


def _sigmoid_body(x_ref, o_ref):
    o_ref[...] = jax.nn.sigmoid(x_ref[...])


def _sigmoid_table(logits):
    return pl.pallas_call(
        _sigmoid_body,
        out_shape=jax.ShapeDtypeStruct((V, D), jnp.float32),
    )(logits)


def _gather_body(s_hbm, idx_hbm, out_hbm, idx_v, buf, sem):
    wid = lax.axis_index("s") * NC + lax.axis_index("c")
    base = wid * BPW
    pltpu.sync_copy(idx_hbm.at[pl.ds(base, BPW)], idx_v)

    def body(j, carry):
        pltpu.async_copy(s_hbm.at[idx_v.at[pl.ds(j * CHUNK, CHUNK)]], buf, sem).wait()
        pltpu.sync_copy(buf, out_hbm.at[pl.ds(base + j * CHUNK, CHUNK)])
        return carry

    lax.fori_loop(0, NCH, body, 0)


_gather = pl.kernel(
    _gather_body,
    out_type=jax.ShapeDtypeStruct((B, D), jnp.float32),
    mesh=plsc.VectorSubcoreMesh(core_axis_name="c", subcore_axis_name="s"),
    scratch_types=[
        pltpu.VMEM((BPW,), jnp.int32),
        pltpu.VMEM((CHUNK, D), jnp.float32),
        pltpu.SemaphoreType.DMA,
    ],
    compiler_params=pltpu.CompilerParams(use_tc_tiling_on_sc=False),
)


@jax.jit
def kernel(indices, logits):
    s = _sigmoid_table(logits)
    return _gather(s, indices)


# speedup vs baseline: 1.1161x; 1.1161x over previous
import jax
import jax.numpy as jnp
from jax import lax
from jax.experimental import pallas as pl
from jax.experimental.pallas import tpu as pltpu
from jax.experimental.pallas import tpu_sc as plsc

B = 16384
V = 1000
D = 1000
DP = 1024
NC = 2
NS = 16
NW = NC * NS
BPW = B // NW
CHUNK = 32
NCH = BPW // CHUNK
RB = 512


def _sigmoid_pad_body(x_ref, o_ref):
    o_ref[:, :D] = jax.nn.sigmoid(x_ref[...])
    o_ref[:, D:] = jnp.zeros((V, DP - D), jnp.float32)


def _sigmoid_table(logits):
    return pl.pallas_call(
        _sigmoid_pad_body,
        out_shape=jax.ShapeDtypeStruct((V, DP), jnp.float32),
    )(logits)


def _gather_body(s_hbm, idx_hbm, out_hbm, idx_v, buf, sem):
    wid = lax.axis_index("s") * NC + lax.axis_index("c")
    base = wid * BPW
    pltpu.sync_copy(idx_hbm.at[pl.ds(base, BPW)], idx_v)

    def body(j, carry):
        pltpu.async_copy(s_hbm.at[idx_v.at[pl.ds(j * CHUNK, CHUNK)]], buf, sem).wait()
        pltpu.sync_copy(buf, out_hbm.at[pl.ds(base + j * CHUNK, CHUNK)])
        return carry

    lax.fori_loop(0, NCH, body, 0)


_gather = pl.kernel(
    _gather_body,
    out_type=jax.ShapeDtypeStruct((B, DP), jnp.float32),
    mesh=plsc.VectorSubcoreMesh(core_axis_name="c", subcore_axis_name="s"),
    scratch_types=[
        pltpu.VMEM((BPW,), jnp.int32),
        pltpu.VMEM((CHUNK, DP), jnp.float32),
        pltpu.SemaphoreType.DMA,
    ],
)


def _unpad_body(x_ref, o_ref):
    o_ref[...] = x_ref[:, :D]


def _unpad(x):
    return pl.pallas_call(
        _unpad_body,
        grid=(B // RB,),
        in_specs=[pl.BlockSpec((RB, DP), lambda i: (i, 0))],
        out_specs=pl.BlockSpec((RB, D), lambda i: (i, 0)),
        out_shape=jax.ShapeDtypeStruct((B, D), jnp.float32),
    )(x)


@jax.jit
def kernel(indices, logits):
    s = _sigmoid_table(logits)
    return _unpad(_gather(s, indices))


# speedup vs baseline: 1.5878x; 1.4227x over previous
import jax
import jax.numpy as jnp
from jax import lax
from jax.experimental import pallas as pl
from jax.experimental.pallas import tpu as pltpu
from jax.experimental.pallas import tpu_sc as plsc

B = 16384
V = 1000
D = 1000
DP = 1024
NC = 2
NS = 16
NW = NC * NS
BPW = B // NW
CHUNK = 32
NCH = BPW // CHUNK
RB = 512


def _sigmoid_pad_body(x_ref, o_ref):
    o_ref[:, :D] = jax.nn.sigmoid(x_ref[...])
    o_ref[:, D:] = jnp.zeros((V, DP - D), jnp.float32)


def _sigmoid_table(logits):
    return pl.pallas_call(
        _sigmoid_pad_body,
        out_shape=jax.ShapeDtypeStruct((V, DP), jnp.float32),
    )(logits)


def _gather_body(s_hbm, idx_hbm, out_hbm, idx_v, buf, sem):
    wid = lax.axis_index("s") * NC + lax.axis_index("c")
    base = wid * BPW
    pltpu.sync_copy(idx_hbm.at[pl.ds(base, BPW)], idx_v)

    def body(j, carry):
        pltpu.async_copy(s_hbm.at[idx_v.at[pl.ds(j * CHUNK, CHUNK)]], buf, sem).wait()
        pltpu.sync_copy(buf, out_hbm.at[pl.ds(base + j * CHUNK, CHUNK)])
        return carry

    lax.fori_loop(0, NCH, body, 0)


_gather = pl.kernel(
    _gather_body,
    out_type=jax.ShapeDtypeStruct((B, DP), jnp.float32),
    mesh=plsc.VectorSubcoreMesh(core_axis_name="c", subcore_axis_name="s"),
    scratch_types=[
        pltpu.VMEM((BPW,), jnp.int32),
        pltpu.VMEM((CHUNK, DP), jnp.float32),
        pltpu.SemaphoreType.DMA,
    ],
)


@jax.jit
def kernel(indices, logits):
    s = _sigmoid_table(logits)
    return _gather(s, indices)[:, :D]


# speedup vs baseline: 1.6519x; 1.0404x over previous
import jax
import jax.numpy as jnp
from jax import lax
from jax.experimental import pallas as pl
from jax.experimental.pallas import tpu as pltpu
from jax.experimental.pallas import tpu_sc as plsc

B = 16384
V = 1000
D = 1000
DP = 1024
NC = 2
NS = 16
NW = NC * NS
BPW = B // NW
CHUNK = 32
NCH = BPW // CHUNK
RB = 512


def _sigmoid_pad_body(x_ref, o_ref):
    o_ref[:, :D] = jax.nn.sigmoid(x_ref[...])
    o_ref[:, D:] = jnp.zeros((V, DP - D), jnp.float32)


def _sigmoid_table(logits):
    return pl.pallas_call(
        _sigmoid_pad_body,
        out_shape=jax.ShapeDtypeStruct((V, DP), jnp.float32),
    )(logits)


def _gather_body(s_hbm, idx_hbm, out_hbm, idx_v, buf0, buf1, si0, si1, so0, so1):
    wid = lax.axis_index("s") * NC + lax.axis_index("c")
    base = wid * BPW
    pltpu.sync_copy(idx_hbm.at[pl.ds(base, BPW)], idx_v)

    bufs = (buf0, buf1)
    sin = (si0, si1)
    sout = (so0, so1)

    def start_in(j, b):
        return pltpu.async_copy(
            s_hbm.at[idx_v.at[pl.ds(j * CHUNK, CHUNK)]], bufs[b], sin[b])

    def start_out(j, b):
        return pltpu.async_copy(
            bufs[b], out_hbm.at[pl.ds(base + j * CHUNK, CHUNK)], sout[b])

    inflight_in = start_in(0, 0)
    inflight_out = [None, None]
    for j in range(NCH):
        b = j % 2
        inflight_in.wait()
        if j + 1 < NCH:
            if inflight_out[1 - b] is not None:
                inflight_out[1 - b].wait()
            inflight_in = start_in(j + 1, 1 - b)
        inflight_out[b] = start_out(j, b)
    for h in inflight_out:
        if h is not None:
            h.wait()


_gather = pl.kernel(
    _gather_body,
    out_type=jax.ShapeDtypeStruct((B, DP), jnp.float32),
    mesh=plsc.VectorSubcoreMesh(core_axis_name="c", subcore_axis_name="s"),
    scratch_types=[
        pltpu.VMEM((BPW,), jnp.int32),
        pltpu.VMEM((CHUNK, DP), jnp.float32),
        pltpu.VMEM((CHUNK, DP), jnp.float32),
        pltpu.SemaphoreType.DMA,
        pltpu.SemaphoreType.DMA,
        pltpu.SemaphoreType.DMA,
        pltpu.SemaphoreType.DMA,
    ],
)


@jax.jit
def kernel(indices, logits):
    s = _sigmoid_table(logits)
    return _gather(s, indices)[:, :D]
